# baseline (speedup 1.0000x reference)
import functools

import jax
import jax.numpy as jnp
from jax import lax
from jax.experimental import pallas as pl
from jax.experimental.pallas import tpu as pltpu
from jax.experimental.pallas import tpu_sc as plsc

N_EMBED = 768
CLF_TOKEN = 0
ROWS = 16384
NC, NS = 2, 16
NW = NC * NS
RW = ROWS // NW
C = 64
G = 16
NG = C // G
NCH = RW // C
NP = NCH // 2
LANES = 16
NJ = N_EMBED // LANES


def _sc_body(h_hbm, x_hbm, w_hbm, b_hbm, out_hbm,
             hb0, hb1, wv, xv, bv, ov, s0, s1):
  cid = lax.axis_index("c")
  sid = lax.axis_index("s")
  wid = sid * NC + cid
  base = wid * RW

  pltpu.sync_copy(w_hbm, wv)
  pltpu.sync_copy(b_hbm, bv)
  pltpu.sync_copy(x_hbm.at[pl.ds(base * 2, RW * 2)], xv.at[pl.ds(0, RW * 2)])
  pltpu.async_copy(h_hbm.at[pl.ds(base, C)], hb0, s0)

  lane = lax.broadcasted_iota(jnp.int32, (LANES,), 0)
  last_lane = lane == (LANES - 1)
  perms = [lane ^ sh for sh in (8, 4, 2, 1)]

  def lane_sum(v):
    for perm in perms:
      v = v + jnp.take_along_axis(v, perm, axis=0, mode="promise_in_bounds")
    return v

  even_perm = (2 * lane) & (LANES - 1)
  SG = 8

  def compute_chunk(buf, chunk_idx):
    def subgroup(rr, lane_base, out_vec):
      accs = [jnp.zeros((LANES,), jnp.float32) for _ in range(SG)]
      for j in range(NJ):
        wj = wv[pl.ds(j * LANES, LANES)]
        for i in range(SG):
          accs[i] = accs[i] + buf[rr + i, pl.ds(j * LANES, LANES)] * wj
      for i in range(SG):
        out_vec = jnp.where(lane == lane_base + i, lane_sum(accs[i]),
                            out_vec)
      return out_vec

    def group(gi, carry):
      rr = gi * LANES
      out_vec = jnp.zeros((LANES,), jnp.float32)
      out_vec = subgroup(rr, 0, out_vec)
      out_vec = subgroup(rr + SG, SG, out_vec)
      rloc = chunk_idx * C + rr
      t0 = xv[pl.ds(2 * rloc, LANES)]
      t1 = xv[pl.ds(2 * rloc + LANES, LANES)]
      e0 = jnp.take_along_axis(t0, even_perm, axis=0,
                               mode="promise_in_bounds")
      e1 = jnp.take_along_axis(t1, even_perm, axis=0,
                               mode="promise_in_bounds")
      tokv = jnp.where(lane < (LANES // 2), e0, e1)
      maskv = jnp.where(tokv == CLF_TOKEN, jnp.float32(1.0),
                        jnp.float32(0.0))
      res = out_vec * maskv + bv[...]
      ov[pl.ds(rloc, LANES)] = res
      return carry

    lax.fori_loop(0, NG, group, 0)

  def start(chunk, buf, sem):
    pltpu.async_copy(h_hbm.at[pl.ds(base + chunk * C, C)], buf, sem)

  def wait(chunk, buf, sem):
    pltpu.make_async_copy(h_hbm.at[pl.ds(base + chunk * C, C)], buf,
                          sem).wait()

  def outer(p, carry):
    c0 = p * 2
    start(c0 + 1, hb1, s1)
    wait(c0, hb0, s0)
    compute_chunk(hb0, c0)

    @pl.when(p < NP - 1)
    def _():
      start(c0 + 2, hb0, s0)

    wait(c0 + 1, hb1, s1)
    compute_chunk(hb1, c0 + 1)
    return carry

  lax.fori_loop(0, NP, outer, 0)
  pltpu.sync_copy(ov, out_hbm.at[pl.ds(base, RW)])


@jax.jit
def _clf_head_sc(hf, xf, wf, bf):
  mesh = plsc.VectorSubcoreMesh(core_axis_name="c", subcore_axis_name="s",
                                num_cores=NC, num_subcores=NS)
  fn = pl.kernel(
      _sc_body,
      out_type=jax.ShapeDtypeStruct((ROWS,), jnp.float32),
      mesh=mesh,
      scratch_types=[
          pltpu.VMEM((C, N_EMBED), jnp.float32),
          pltpu.VMEM((C, N_EMBED), jnp.float32),
          pltpu.VMEM((N_EMBED,), jnp.float32),
          pltpu.VMEM((RW * 2 + LANES,), jnp.int32),
          pltpu.VMEM((LANES,), jnp.float32),
          pltpu.VMEM((RW,), jnp.float32),
          pltpu.SemaphoreType.DMA,
          pltpu.SemaphoreType.DMA,
      ],
  )
  return fn(hf, xf, wf, bf)


def kernel(h, x, W, b):
  hf = h.reshape(ROWS, N_EMBED)
  xf = x.reshape(ROWS * 2)
  wf = W.reshape(N_EMBED).astype(jnp.float32)
  bf = jnp.tile(b.astype(jnp.float32), LANES)
  out = _clf_head_sc(hf, xf, wf, bf)
  return out.reshape(-1, 2)


# speedup vs baseline: 2.9035x; 2.9035x over previous
import jax
import jax.numpy as jnp
from jax import lax
from jax.experimental import pallas as pl
from jax.experimental.pallas import tpu as pltpu
from jax.experimental.pallas import tpu_sc as plsc

N_EMBED = 768
CLF_TOKEN = 0
ROWS = 16384
SEQ = 2048
NC, NS = 2, 16
NW = NC * NS
RWS = 128
SC_ROWS = NW * RWS
RT = ROWS - SC_ROWS
C = 64
NCH = RWS // C
NP = NCH // 2
LANES = 16
NJ = N_EMBED // LANES
BR = 1024


def _sc_body(h_hbm, x_hbm, w_hbm, b_hbm, out_hbm,
             hb0, hb1, wv, xv, bv, ov, s0, s1):
  cid = lax.axis_index("c")
  sid = lax.axis_index("s")
  wid = sid * NC + cid
  gbase = RT + wid * RWS
  bidx = gbase // SEQ
  roff = gbase % SEQ

  pltpu.sync_copy(w_hbm, wv)
  pltpu.sync_copy(b_hbm, bv)
  pltpu.sync_copy(x_hbm.at[bidx, pl.ds(roff, RWS)], xv)
  pltpu.async_copy(h_hbm.at[bidx, pl.ds(roff, C), :], hb0, s0)

  lane = lax.broadcasted_iota(jnp.int32, (LANES,), 0)
  perms = [lane ^ sh for sh in (8, 4, 2, 1)]

  def lane_sum(v):
    for perm in perms:
      v = v + jnp.take_along_axis(v, perm, axis=0, mode="promise_in_bounds")
    return v

  zero16 = jnp.zeros((LANES,), jnp.float32)

  def compute_chunk(buf, chunk_idx):
    def group(gi, carry):
      rr = gi * LANES
      @plsc.parallel_loop(0, NJ, unroll=2,
                          carry=tuple(zero16 for _ in range(LANES)))
      def accs(j, acc):
        off = j * LANES
        wj = wv[pl.ds(off, LANES)]
        return tuple(
            acc[i] + buf[rr + i, pl.ds(off, LANES)] * wj
            for i in range(LANES))

      out_vec = lane_sum(accs[0])
      for i in range(1, LANES):
        out_vec = jnp.where(lane == i, lane_sum(accs[i]), out_vec)
      rloc = chunk_idx * C + rr
      tokv = xv[pl.ds(rloc, LANES)]
      maskv = jnp.where(tokv == CLF_TOKEN, jnp.float32(1.0),
                        jnp.float32(0.0))
      res = out_vec * maskv + bv[...]
      ov[pl.ds(rloc, LANES)] = res
      return carry

    lax.fori_loop(0, C // LANES, group, 0)

  def start(chunk, buf, sem):
    pltpu.async_copy(h_hbm.at[bidx, pl.ds(roff + chunk * C, C), :], buf, sem)

  def wait(chunk, buf, sem):
    pltpu.make_async_copy(h_hbm.at[bidx, pl.ds(roff + chunk * C, C), :], buf,
                          sem).wait()

  for p in range(NP):
    c0 = p * 2
    start(c0 + 1, hb1, s1)
    wait(c0, hb0, s0)
    compute_chunk(hb0, c0)
    if p < NP - 1:
      start(c0 + 2, hb0, s0)
    wait(c0 + 1, hb1, s1)
    compute_chunk(hb1, c0 + 1)

  pltpu.sync_copy(ov, out_hbm.at[pl.ds(wid * RWS, RWS)])


def _tc_body(x_ref, h_ref, w_ref, b_ref, out_ref):
  hv = h_ref[...]
  wv = w_ref[...]
  dots = jnp.sum(hv * wv[None, :], axis=1)
  maskf = jnp.where(x_ref[...] == CLF_TOKEN, jnp.float32(1.0),
                    jnp.float32(0.0))
  out_ref[...] = dots * maskf + b_ref[0]


@jax.jit
def _clf_head(h, xt, wf, bf):
  mesh = plsc.VectorSubcoreMesh(core_axis_name="c", subcore_axis_name="s",
                                num_cores=NC, num_subcores=NS)
  sc_fn = pl.kernel(
      _sc_body,
      out_type=jax.ShapeDtypeStruct((SC_ROWS,), jnp.float32),
      mesh=mesh,
      scratch_types=[
          pltpu.VMEM((C, N_EMBED), jnp.float32),
          pltpu.VMEM((C, N_EMBED), jnp.float32),
          pltpu.VMEM((N_EMBED,), jnp.float32),
          pltpu.VMEM((RWS,), jnp.int32),
          pltpu.VMEM((LANES,), jnp.float32),
          pltpu.VMEM((RWS,), jnp.float32),
          pltpu.SemaphoreType.DMA,
          pltpu.SemaphoreType.DMA,
      ],
  )
  out_sc = sc_fn(h, xt, wf, bf)

  hf = h.reshape(ROWS, N_EMBED)
  xf = xt.reshape(ROWS)
  out_tc = pl.pallas_call(
      _tc_body,
      grid=(RT // BR,),
      in_specs=[
          pl.BlockSpec((BR,), lambda i: (i,)),
          pl.BlockSpec((BR, N_EMBED), lambda i: (i, 0)),
          pl.BlockSpec((N_EMBED,), lambda i: (0,)),
          pl.BlockSpec((LANES,), lambda i: (0,)),
      ],
      out_specs=pl.BlockSpec((BR,), lambda i: (i,)),
      out_shape=jax.ShapeDtypeStruct((RT,), jnp.float32),
  )(xf, hf, wf, bf)

  return jnp.concatenate([out_tc, out_sc]).reshape(-1, 2)


def kernel(h, x, W, b):
  xt = x[:, :, 0]
  wf = W.reshape(N_EMBED).astype(jnp.float32)
  bf = jnp.tile(b.astype(jnp.float32), LANES)
  return _clf_head(h, xt, wf, bf)
